# Initial kernel scaffold; baseline (speedup 1.0000x reference)
import functools

import jax
import jax.numpy as jnp
from jax import lax
from jax.experimental import pallas as pl
from jax.experimental.pallas import tpu as pltpu
from jax.experimental.pallas import tpu_sc as plsc

_NC = 2
_NS = 16
_NW = _NC * _NS
_L = 16

_SPLIT = 128


def _pooled_gather_body(n_per_w, seq, dim, idx_hbm, table_hbm, out_hbm,
                        ia0, ib0, ia1, ib1, r0, r1, out_buf,
                        si0, si1, sg0, sg1):
  rem = seq - _SPLIT
  wid = lax.axis_index("s") * _NC + lax.axis_index("c")
  base = wid * n_per_w

  def fetch_idx(i_local, ia, ib, sem):
    row = base + lax.rem(i_local, n_per_w)
    pltpu.async_copy(idx_hbm.at[row, pl.ds(0, _SPLIT)], ia, sem)
    pltpu.async_copy(idx_hbm.at[row, pl.ds(_SPLIT, rem)], ib, sem)

  def wait_idx(ia, ib, sem):
    pltpu.make_async_copy(idx_hbm.at[0, pl.ds(0, _SPLIT)], ia, sem).wait()
    pltpu.make_async_copy(idx_hbm.at[0, pl.ds(_SPLIT, rem)], ib, sem).wait()

  def start_gather(ia, ib, r, sem):
    pltpu.async_copy(table_hbm.at[ia], r.at[pl.ds(0, _SPLIT)], sem)
    pltpu.async_copy(table_hbm.at[ib], r.at[pl.ds(_SPLIT, rem)], sem)

  def wait_gather(ia, ib, r, sem):
    pltpu.make_async_copy(table_hbm.at[ia], r.at[pl.ds(0, _SPLIT)], sem).wait()
    pltpu.make_async_copy(table_hbm.at[ib], r.at[pl.ds(_SPLIT, rem)], sem).wait()

  nacc = dim // _L
  unroll = 4

  def reduce(r, i_local):
    def body(j, accs):
      rr = j * unroll
      out = []
      for c in range(nacc):
        sl = pl.ds(c * _L, _L)
        x0 = r[rr, sl]
        x1 = r[rr + 1, sl]
        x2 = r[rr + 2, sl]
        x3 = r[rr + 3, sl]
        out.append(accs[c] + ((x0 + x1) + (x2 + x3)))
      return tuple(out)

    zero = jnp.zeros((_L,), jnp.float32)
    accs = lax.fori_loop(0, seq // unroll, body, (zero,) * nacc)
    for c in range(nacc):
      out_buf[i_local, pl.ds(c * _L, _L)] = accs[c]

  fetch_idx(0, ia0, ib0, si0)
  fetch_idx(1, ia1, ib1, si1)
  wait_idx(ia0, ib0, si0)
  start_gather(ia0, ib0, r0, sg0)

  def outer(g, carry):
    i0 = 2 * g
    i1 = i0 + 1
    wait_idx(ia1, ib1, si1)
    start_gather(ia1, ib1, r1, sg1)
    wait_gather(ia0, ib0, r0, sg0)
    fetch_idx(i0 + 2, ia0, ib0, si0)
    reduce(r0, i0)
    wait_idx(ia0, ib0, si0)
    start_gather(ia0, ib0, r0, sg0)
    wait_gather(ia1, ib1, r1, sg1)
    fetch_idx(i1 + 2, ia1, ib1, si1)
    reduce(r1, i1)
    return carry

  lax.fori_loop(0, n_per_w // 2, outer, 0)

  wait_gather(ia0, ib0, r0, sg0)
  wait_idx(ia1, ib1, si1)

  pltpu.sync_copy(out_buf, out_hbm.at[pl.ds(base, n_per_w)])


@functools.lru_cache(maxsize=None)
def _build(rows, seq, dim):
  assert rows % _NW == 0 and dim % _L == 0 and seq % 4 == 0
  n_per_w = rows // _NW
  rem = seq - _SPLIT
  assert 0 < rem <= _SPLIT and _SPLIT % 8 == 0

  mesh = plsc.VectorSubcoreMesh(
      core_axis_name="c", subcore_axis_name="s",
      num_cores=_NC, num_subcores=_NS)
  return pl.kernel(
      functools.partial(_pooled_gather_body, n_per_w, seq, dim),
      out_type=jax.ShapeDtypeStruct((rows, dim), jnp.float32),
      mesh=mesh,
      scratch_types=[
          pltpu.VMEM((_SPLIT,), jnp.int32),
          pltpu.VMEM((rem,), jnp.int32),
          pltpu.VMEM((_SPLIT,), jnp.int32),
          pltpu.VMEM((rem,), jnp.int32),
          pltpu.VMEM((seq, dim), jnp.float32),
          pltpu.VMEM((seq, dim), jnp.float32),
          pltpu.VMEM((n_per_w, dim), jnp.float32),
          pltpu.SemaphoreType.DMA,
          pltpu.SemaphoreType.DMA,
          pltpu.SemaphoreType.DMA,
          pltpu.SemaphoreType.DMA,
      ],
  )


def kernel(input_a, input_b, embedding_matrix):
  b, seq = input_a.shape
  dim = embedding_matrix.shape[1]
  idx = jnp.concatenate([input_a, input_b], axis=0).astype(jnp.int32)
  out = _build(2 * b, seq, dim)(idx, embedding_matrix)
  return out[:b], out[b:]


# speedup vs baseline: 1.2646x; 1.2646x over previous
import functools

import jax
import jax.numpy as jnp
from jax import lax
from jax.experimental import pallas as pl
from jax.experimental.pallas import tpu as pltpu
from jax.experimental.pallas import tpu_sc as plsc

_NC = 2
_NS = 16
_NW = _NC * _NS
_L = 16

_SPLIT = 128


def _pooled_gather_body(n_per_w, seq, dim, idx_hbm, table_hbm, out_hbm,
                        ia0, ib0, ia1, ib1, r0, r1, out_buf,
                        si0, si1, sg0, sg1):
  rem = seq - _SPLIT
  wid = lax.axis_index("s") * _NC + lax.axis_index("c")
  base = wid * n_per_w

  def fetch_idx(i_local, ia, ib, sem):
    row = base + lax.rem(i_local, n_per_w)
    pltpu.async_copy(idx_hbm.at[row, pl.ds(0, _SPLIT)], ia, sem)
    pltpu.async_copy(idx_hbm.at[row, pl.ds(_SPLIT, rem)], ib, sem)

  def wait_idx(ia, ib, sem):
    pltpu.make_async_copy(idx_hbm.at[0, pl.ds(0, _SPLIT)], ia, sem).wait()
    pltpu.make_async_copy(idx_hbm.at[0, pl.ds(_SPLIT, rem)], ib, sem).wait()

  def start_gather(ia, ib, r, sem):
    pltpu.async_copy(table_hbm.at[ia], r.at[pl.ds(0, _SPLIT)], sem)
    pltpu.async_copy(table_hbm.at[ib], r.at[pl.ds(_SPLIT, rem)], sem)

  def wait_gather(ia, ib, r, sem):
    pltpu.make_async_copy(table_hbm.at[ia], r.at[pl.ds(0, _SPLIT)], sem).wait()
    pltpu.make_async_copy(table_hbm.at[ib], r.at[pl.ds(_SPLIT, rem)], sem).wait()

  nacc = dim // _L
  unroll = 4

  def reduce(r, i_local):
    def body(j, accs):
      rr = j * unroll
      out = []
      for c in range(nacc):
        sl = pl.ds(c * _L, _L)
        x0 = r[rr, sl]
        x1 = r[rr + 1, sl]
        x2 = r[rr + 2, sl]
        x3 = r[rr + 3, sl]
        out.append(accs[c] + ((x0 + x1) + (x2 + x3)))
      return tuple(out)

    zero = jnp.zeros((_L,), jnp.float32)
    accs = lax.fori_loop(0, seq // unroll, body, (zero,) * nacc)
    for c in range(nacc):
      out_buf[i_local, pl.ds(c * _L, _L)] = accs[c]

  fetch_idx(0, ia0, ib0, si0)
  fetch_idx(1, ia1, ib1, si1)
  wait_idx(ia0, ib0, si0)
  start_gather(ia0, ib0, r0, sg0)

  def outer(g, carry):
    i0 = 2 * g
    i1 = i0 + 1
    wait_idx(ia1, ib1, si1)
    start_gather(ia1, ib1, r1, sg1)
    wait_gather(ia0, ib0, r0, sg0)
    fetch_idx(i0 + 2, ia0, ib0, si0)
    reduce(r0, i0)
    wait_idx(ia0, ib0, si0)
    start_gather(ia0, ib0, r0, sg0)
    wait_gather(ia1, ib1, r1, sg1)
    fetch_idx(i1 + 2, ia1, ib1, si1)
    reduce(r1, i1)
    return carry

  lax.fori_loop(0, n_per_w // 2, outer, 0)

  wait_gather(ia0, ib0, r0, sg0)
  wait_idx(ia1, ib1, si1)

  pltpu.sync_copy(out_buf, out_hbm.at[pl.ds(base, n_per_w)])


@functools.lru_cache(maxsize=None)
def _build(rows, seq, dim):
  assert rows % _NW == 0 and dim % _L == 0 and seq % 4 == 0
  n_per_w = rows // _NW
  rem = seq - _SPLIT
  assert 0 < rem <= _SPLIT and _SPLIT % 8 == 0

  mesh = plsc.VectorSubcoreMesh(
      core_axis_name="c", subcore_axis_name="s",
      num_cores=_NC, num_subcores=_NS)
  return pl.kernel(
      functools.partial(_pooled_gather_body, n_per_w, seq, dim),
      out_type=jax.ShapeDtypeStruct((rows, dim), jnp.float32),
      mesh=mesh,
      compiler_params=pltpu.CompilerParams(use_tc_tiling_on_sc=False),
      scratch_types=[
          pltpu.VMEM((_SPLIT,), jnp.int32),
          pltpu.VMEM((rem,), jnp.int32),
          pltpu.VMEM((_SPLIT,), jnp.int32),
          pltpu.VMEM((rem,), jnp.int32),
          pltpu.VMEM((seq, dim), jnp.float32),
          pltpu.VMEM((seq, dim), jnp.float32),
          pltpu.VMEM((n_per_w, dim), jnp.float32),
          pltpu.SemaphoreType.DMA,
          pltpu.SemaphoreType.DMA,
          pltpu.SemaphoreType.DMA,
          pltpu.SemaphoreType.DMA,
      ],
  )


def kernel(input_a, input_b, embedding_matrix):
  b, seq = input_a.shape
  dim = embedding_matrix.shape[1]
  idx = jnp.concatenate([input_a, input_b], axis=0).astype(jnp.int32)
  out = _build(2 * b, seq, dim)(idx, embedding_matrix)
  return out[:b], out[b:]


# speedup vs baseline: 1.2784x; 1.0109x over previous
import functools

import jax
import jax.numpy as jnp
from jax import lax
from jax.experimental import pallas as pl
from jax.experimental.pallas import tpu as pltpu
from jax.experimental.pallas import tpu_sc as plsc

_NC = 2
_NS = 16
_NW = _NC * _NS
_L = 16

_SPLIT = 128


def _pooled_gather_body(n_per_w, seq, dim, idx_a, idx_b, table_hbm,
                        out_a, out_b,
                        ia0, ib0, ia1, ib1, r0, r1, out_buf,
                        si0, si1, sg0, sg1):
  rem = seq - _SPLIT
  wid = lax.axis_index("s") * _NC + lax.axis_index("c")
  nacc = dim // _L
  unroll = 4

  def run_pipeline(idx_hbm, out_hbm, base):
    def fetch_idx(i_local, ia, ib, sem):
      row = base + lax.rem(i_local, n_per_w)
      pltpu.async_copy(idx_hbm.at[row, pl.ds(0, _SPLIT)], ia, sem)
      pltpu.async_copy(idx_hbm.at[row, pl.ds(_SPLIT, rem)], ib, sem)

    def wait_idx(ia, ib, sem):
      pltpu.make_async_copy(idx_hbm.at[0, pl.ds(0, _SPLIT)], ia, sem).wait()
      pltpu.make_async_copy(idx_hbm.at[0, pl.ds(_SPLIT, rem)], ib, sem).wait()

    def start_gather(ia, ib, r, sem):
      pltpu.async_copy(table_hbm.at[ia], r.at[pl.ds(0, _SPLIT)], sem)
      pltpu.async_copy(table_hbm.at[ib], r.at[pl.ds(_SPLIT, rem)], sem)

    def wait_gather(ia, ib, r, sem):
      pltpu.make_async_copy(table_hbm.at[ia], r.at[pl.ds(0, _SPLIT)], sem).wait()
      pltpu.make_async_copy(table_hbm.at[ib], r.at[pl.ds(_SPLIT, rem)],
                            sem).wait()

    def reduce(r, i_local):
      def body(j, accs):
        rr = j * unroll
        out = []
        for c in range(nacc):
          sl = pl.ds(c * _L, _L)
          x0 = r[rr, sl]
          x1 = r[rr + 1, sl]
          x2 = r[rr + 2, sl]
          x3 = r[rr + 3, sl]
          out.append(accs[c] + ((x0 + x1) + (x2 + x3)))
        return tuple(out)

      zero = jnp.zeros((_L,), jnp.float32)
      accs = lax.fori_loop(0, seq // unroll, body, (zero,) * nacc)
      for c in range(nacc):
        out_buf[i_local, pl.ds(c * _L, _L)] = accs[c]

    fetch_idx(0, ia0, ib0, si0)
    fetch_idx(1, ia1, ib1, si1)
    wait_idx(ia0, ib0, si0)
    start_gather(ia0, ib0, r0, sg0)

    def outer(g, carry):
      i0 = 2 * g
      i1 = i0 + 1
      wait_idx(ia1, ib1, si1)
      start_gather(ia1, ib1, r1, sg1)
      wait_gather(ia0, ib0, r0, sg0)
      fetch_idx(i0 + 2, ia0, ib0, si0)
      reduce(r0, i0)
      wait_idx(ia0, ib0, si0)
      start_gather(ia0, ib0, r0, sg0)
      wait_gather(ia1, ib1, r1, sg1)
      fetch_idx(i1 + 2, ia1, ib1, si1)
      reduce(r1, i1)
      return carry

    lax.fori_loop(0, n_per_w // 2, outer, 0)

    wait_gather(ia0, ib0, r0, sg0)
    wait_idx(ia1, ib1, si1)

    pltpu.sync_copy(out_buf, out_hbm.at[pl.ds(base, n_per_w)])

  @pl.when(wid < _NS)
  def _():
    run_pipeline(idx_a, out_a, wid * n_per_w)

  @pl.when(wid >= _NS)
  def _():
    run_pipeline(idx_b, out_b, (wid - _NS) * n_per_w)


@functools.lru_cache(maxsize=None)
def _build(batch, seq, dim):
  assert batch % _NS == 0 and dim % _L == 0 and seq % 4 == 0
  n_per_w = batch // _NS
  rem = seq - _SPLIT
  assert 0 < rem <= _SPLIT and _SPLIT % 8 == 0

  mesh = plsc.VectorSubcoreMesh(
      core_axis_name="c", subcore_axis_name="s",
      num_cores=_NC, num_subcores=_NS)
  out_t = jax.ShapeDtypeStruct((batch, dim), jnp.float32)
  return pl.kernel(
      functools.partial(_pooled_gather_body, n_per_w, seq, dim),
      out_type=(out_t, out_t),
      mesh=mesh,
      compiler_params=pltpu.CompilerParams(use_tc_tiling_on_sc=False),
      scratch_types=[
          pltpu.VMEM((_SPLIT,), jnp.int32),
          pltpu.VMEM((rem,), jnp.int32),
          pltpu.VMEM((_SPLIT,), jnp.int32),
          pltpu.VMEM((rem,), jnp.int32),
          pltpu.VMEM((seq, dim), jnp.float32),
          pltpu.VMEM((seq, dim), jnp.float32),
          pltpu.VMEM((n_per_w, dim), jnp.float32),
          pltpu.SemaphoreType.DMA,
          pltpu.SemaphoreType.DMA,
          pltpu.SemaphoreType.DMA,
          pltpu.SemaphoreType.DMA,
      ],
  )


def kernel(input_a, input_b, embedding_matrix):
  b, seq = input_a.shape
  dim = embedding_matrix.shape[1]
  return _build(b, seq, dim)(input_a.astype(jnp.int32),
                             input_b.astype(jnp.int32),
                             embedding_matrix)


# speedup vs baseline: 1.4037x; 1.0980x over previous
import functools

import jax
import jax.numpy as jnp
from jax import lax
from jax.experimental import pallas as pl
from jax.experimental.pallas import tpu as pltpu
from jax.experimental.pallas import tpu_sc as plsc

_NC = 2
_NS = 16
_NW = _NC * _NS
_L = 16

_SPLIT = 128


_NBUF = 4


def _pooled_gather_body(n_per_w, seq, dim, idx_a, idx_b, table_hbm,
                        out_a, out_b, *scratch):
  ias = scratch[0:_NBUF]
  ibs = scratch[_NBUF:2 * _NBUF]
  rs = scratch[2 * _NBUF:3 * _NBUF]
  out_buf = scratch[3 * _NBUF]
  sis = scratch[3 * _NBUF + 1:3 * _NBUF + 1 + _NBUF]
  sgs = scratch[3 * _NBUF + 1 + _NBUF:3 * _NBUF + 1 + 2 * _NBUF]
  rem = seq - _SPLIT
  wid = lax.axis_index("s") * _NC + lax.axis_index("c")
  nacc = dim // _L
  unroll = 4

  def run_pipeline(idx_hbm, out_hbm, base):
    def fetch_idx(i_local, ia, ib, sem):
      row = base + lax.rem(i_local, n_per_w)
      pltpu.async_copy(idx_hbm.at[row, pl.ds(0, _SPLIT)], ia, sem)
      pltpu.async_copy(idx_hbm.at[row, pl.ds(_SPLIT, rem)], ib, sem)

    def wait_idx(ia, ib, sem):
      pltpu.make_async_copy(idx_hbm.at[0, pl.ds(0, _SPLIT)], ia, sem).wait()
      pltpu.make_async_copy(idx_hbm.at[0, pl.ds(_SPLIT, rem)], ib, sem).wait()

    def start_gather(ia, ib, r, sem):
      pltpu.async_copy(table_hbm.at[ia], r.at[pl.ds(0, _SPLIT)], sem)
      pltpu.async_copy(table_hbm.at[ib], r.at[pl.ds(_SPLIT, rem)], sem)

    def wait_gather(ia, ib, r, sem):
      pltpu.make_async_copy(table_hbm.at[ia], r.at[pl.ds(0, _SPLIT)], sem).wait()
      pltpu.make_async_copy(table_hbm.at[ib], r.at[pl.ds(_SPLIT, rem)],
                            sem).wait()

    def reduce(r, i_local):
      def body(j, accs):
        rr = j * unroll
        out = []
        for c in range(nacc):
          sl = pl.ds(c * _L, _L)
          x0 = r[rr, sl]
          x1 = r[rr + 1, sl]
          x2 = r[rr + 2, sl]
          x3 = r[rr + 3, sl]
          out.append(accs[c] + ((x0 + x1) + (x2 + x3)))
        return tuple(out)

      zero = jnp.zeros((_L,), jnp.float32)
      accs = lax.fori_loop(0, seq // unroll, body, (zero,) * nacc)
      for c in range(nacc):
        out_buf[i_local, pl.ds(c * _L, _L)] = accs[c]

    for k in range(_NBUF):
      fetch_idx(k, ias[k], ibs[k], sis[k])
    for k in range(_NBUF - 1):
      wait_idx(ias[k], ibs[k], sis[k])
      start_gather(ias[k], ibs[k], rs[k], sgs[k])

    def outer(g, carry):
      i0 = _NBUF * g
      for k in range(_NBUF):
        t = i0 + k
        bn = (k + _NBUF - 1) % _NBUF
        wait_gather(ias[k], ibs[k], rs[k], sgs[k])
        fetch_idx(t + _NBUF, ias[k], ibs[k], sis[k])
        wait_idx(ias[bn], ibs[bn], sis[bn])
        start_gather(ias[bn], ibs[bn], rs[bn], sgs[bn])
        reduce(rs[k], t)
      return carry

    lax.fori_loop(0, n_per_w // _NBUF, outer, 0)

    for k in range(_NBUF - 1):
      wait_gather(ias[k], ibs[k], rs[k], sgs[k])
    wait_idx(ias[_NBUF - 1], ibs[_NBUF - 1], sis[_NBUF - 1])

    pltpu.sync_copy(out_buf, out_hbm.at[pl.ds(base, n_per_w)])

  @pl.when(wid < _NS)
  def _():
    run_pipeline(idx_a, out_a, wid * n_per_w)

  @pl.when(wid >= _NS)
  def _():
    run_pipeline(idx_b, out_b, (wid - _NS) * n_per_w)


@functools.lru_cache(maxsize=None)
def _build(batch, seq, dim):
  assert batch % _NS == 0 and dim % _L == 0 and seq % 4 == 0
  n_per_w = batch // _NS
  assert n_per_w % _NBUF == 0
  rem = seq - _SPLIT
  assert 0 < rem <= _SPLIT and _SPLIT % 8 == 0

  mesh = plsc.VectorSubcoreMesh(
      core_axis_name="c", subcore_axis_name="s",
      num_cores=_NC, num_subcores=_NS)
  out_t = jax.ShapeDtypeStruct((batch, dim), jnp.float32)
  return pl.kernel(
      functools.partial(_pooled_gather_body, n_per_w, seq, dim),
      out_type=(out_t, out_t),
      mesh=mesh,
      compiler_params=pltpu.CompilerParams(use_tc_tiling_on_sc=False),
      scratch_types=(
          [pltpu.VMEM((_SPLIT,), jnp.int32) for _ in range(_NBUF)]
          + [pltpu.VMEM((rem,), jnp.int32) for _ in range(_NBUF)]
          + [pltpu.VMEM((seq, dim), jnp.float32) for _ in range(_NBUF)]
          + [pltpu.VMEM((n_per_w, dim), jnp.float32)]
          + [pltpu.SemaphoreType.DMA for _ in range(2 * _NBUF)]
      ),
  )


def kernel(input_a, input_b, embedding_matrix):
  b, seq = input_a.shape
  dim = embedding_matrix.shape[1]
  return _build(b, seq, dim)(input_a.astype(jnp.int32),
                             input_b.astype(jnp.int32),
                             embedding_matrix)
